# baseline (speedup 1.0000x reference)
import functools

import jax
import jax.numpy as jnp
from jax import lax
from jax.experimental import pallas as pl
from jax.experimental.pallas import tpu as pltpu
from jax.experimental.pallas import tpu_sc as plsc

_info = plsc.get_sparse_core_info()
_NC, _NS, _L = _info.num_cores, _info.num_subcores, _info.num_lanes
_NW = _NC * _NS

_V = 64
_E = 64
_GPAD = 8
_TOK = 1024 * 20
_TPT = _TOK // _NW
_CHUNK = 128
_NCHUNK = _TPT // _CHUNK

_mesh = plsc.VectorSubcoreMesh(core_axis_name="c", subcore_axis_name="s")
_sc_params = pltpu.CompilerParams(use_tc_tiling_on_sc=False)


def _build_emb_body(blk_ids, lane_ids, tt_blk, cntb_blk, emb_blk, acc, emb_acc):
    w = pl.program_id(0)
    k = pl.program_id(1)
    lane = lane_ids[w * _GPAD + k]
    iota = lax.broadcasted_iota(jnp.int32, (_E, 128), 1)
    contrib = jnp.where(iota == lane, tt_blk[...], 0.0)

    @pl.when(k == 0)
    def _():
        acc[...] = contrib

    @pl.when(k > 0)
    def _():
        acc[...] = acc[...] + contrib

    @pl.when(k == _GPAD - 1)
    def _():
        ones = jnp.ones((1, 128), jnp.float32)
        row = lax.dot_general(ones, acc[...], (((1,), (1,)), ((), ())),
                              preferred_element_type=jnp.float32)
        wmask = lax.broadcasted_iota(jnp.int32, (_V, _E), 0) == w
        upd = jnp.where(wmask, jnp.broadcast_to(row, (_V, _E)), 0.0)

        @pl.when(w == 0)
        def _():
            emb_acc[...] = upd

        @pl.when(w > 0)
        def _():
            emb_acc[...] = emb_acc[...] + upd

        @pl.when(w == _V - 1)
        def _():
            emb_blk[...] = emb_acc[...] / cntb_blk[...]


_build_emb = pl.pallas_call(
    _build_emb_body,
    grid_spec=pltpu.PrefetchScalarGridSpec(
        num_scalar_prefetch=2,
        grid=(_V, _GPAD),
        in_specs=[
            pl.BlockSpec((_E, 128), lambda w, k, blk, lane: (0, blk[w * _GPAD + k])),
            pl.BlockSpec((_V, _E), lambda w, k, blk, lane: (0, 0)),
        ],
        out_specs=pl.BlockSpec((_V, _E), lambda w, k, blk, lane: (0, 0)),
        scratch_shapes=[
            pltpu.VMEM((_E, 128), jnp.float32),
            pltpu.VMEM((_V, _E), jnp.float32),
        ],
    ),
    out_shape=jax.ShapeDtypeStruct((_V, _E), jnp.float32),
)


@functools.partial(
    pl.kernel,
    mesh=_mesh,
    compiler_params=_sc_params,
    out_type=jax.ShapeDtypeStruct((_TOK, _E), jnp.float32),
    scratch_types=[
        pltpu.VMEM((_NCHUNK, _CHUNK), jnp.int32),
        pltpu.VMEM((_TPT, _E), jnp.float32),
        pltpu.SemaphoreType.DMA,
    ],
)
def _expand(emb_hbm, idx_hbm, out_hbm, idx_v, rows_v, sem):
    wid = lax.axis_index("s") * _NC + lax.axis_index("c")
    pltpu.sync_copy(idx_hbm.at[wid], idx_v)
    copies = []
    for j in range(_NCHUNK):
        copies.append(
            pltpu.async_copy(
                emb_hbm.at[idx_v.at[j]],
                rows_v.at[pl.ds(j * _CHUNK, _CHUNK)],
                sem,
            )
        )
    for c in copies:
        c.wait()
    pltpu.sync_copy(rows_v, out_hbm.at[pl.ds(wid * _TPT, _TPT)])


def kernel(word_idx, table, ngram_idx, ngram_cnt):
    tt = table.T
    idxp = jnp.pad(ngram_idx, ((0, 0), (0, _GPAD - ngram_idx.shape[1])))
    idxf = idxp.reshape(_V * _GPAD)
    blk_ids = idxf // 128
    lane_ids = idxf % 128
    cntb = jnp.broadcast_to(ngram_cnt[:, None], (_V, _E))
    emb = _build_emb(blk_ids, lane_ids, tt, cntb)
    tok_idx = word_idx.reshape(_NW, _NCHUNK, _CHUNK)
    out = _expand(emb, tok_idx)
    return out.reshape(word_idx.shape + (_E,))


# speedup vs baseline: 2.0581x; 2.0581x over previous
import functools

import jax
import jax.numpy as jnp
from jax import lax
from jax.experimental import pallas as pl
from jax.experimental.pallas import tpu as pltpu
from jax.experimental.pallas import tpu_sc as plsc

_info = plsc.get_sparse_core_info()
_NC, _NS, _L = _info.num_cores, _info.num_subcores, _info.num_lanes
_NW = _NC * _NS

_B = 1024
_S = 20
_V = 64
_E = 64
_GPAD = 8
_WPS = _V // _NS
_TOK = _B * _S
_TPT = _TOK // _NW
_CHUNK = 128
_NCHUNK = _TPT // _CHUNK

_mesh = plsc.VectorSubcoreMesh(core_axis_name="c", subcore_axis_name="s")
_sc_params = pltpu.CompilerParams(use_tc_tiling_on_sc=False)


@functools.partial(
    pl.kernel,
    mesh=_mesh,
    compiler_params=_sc_params,
    out_type=(
        jax.ShapeDtypeStruct((_NC, _V, _E), jnp.float32),
        jax.ShapeDtypeStruct((_TOK, _E), jnp.float32),
    ),
    scratch_types=[
        pltpu.VMEM((_WPS * _GPAD,), jnp.int32),
        pltpu.VMEM((_WPS * _GPAD, _E), jnp.float32),
        pltpu.VMEM((_WPS, _E), jnp.float32),
        pltpu.VMEM((_WPS, _E), jnp.float32),
        pltpu.VMEM((_NCHUNK, _CHUNK), jnp.int32),
        pltpu.VMEM((_TPT, _E), jnp.float32),
        pltpu.SemaphoreType.DMA,
        pltpu.SemaphoreType.DMA,
    ],
)
def _ngram_embed(table_hbm, idxp_hbm, cntb_hbm, tok_hbm, emb_hbm, out_hbm,
                 idxA_v, rowsA_v, cnt_v, emb_v, idx_v, rows_v, semA, semB):
    cid = lax.axis_index("c")
    sid = lax.axis_index("s")
    wid = sid * _NC + cid
    cp_idx = pltpu.async_copy(tok_hbm.at[wid], idx_v, semB)
    pltpu.sync_copy(idxp_hbm.at[sid], idxA_v)
    pltpu.async_copy(table_hbm.at[idxA_v], rowsA_v, semA).wait()
    pltpu.sync_copy(cntb_hbm.at[pl.ds(sid * _WPS, _WPS)], cnt_v)
    for wloc in range(_WPS):
        for c in range(_E // _L):
            acc = rowsA_v[_GPAD * wloc, pl.ds(c * _L, _L)]
            for l in range(1, _GPAD):
                acc = acc + rowsA_v[_GPAD * wloc + l, pl.ds(c * _L, _L)]
            emb_v[wloc, pl.ds(c * _L, _L)] = acc / cnt_v[wloc, pl.ds(c * _L, _L)]
    pltpu.sync_copy(emb_v, emb_hbm.at[cid].at[pl.ds(sid * _WPS, _WPS)])
    plsc.subcore_barrier()
    cp_idx.wait()
    copies = []
    for j in range(_NCHUNK):
        copies.append(
            pltpu.async_copy(
                emb_hbm.at[cid].at[idx_v.at[j]],
                rows_v.at[pl.ds(j * _CHUNK, _CHUNK)],
                semA,
            )
        )
    for c in copies:
        c.wait()
    pltpu.sync_copy(rows_v, out_hbm.at[pl.ds(wid * _TPT, _TPT)])


def _fmt_body(src_blk, dst_blk):
    dst_blk[0] = jnp.transpose(src_blk[0], (1, 0))


_fmt = pl.pallas_call(
    _fmt_body,
    grid=(_S,),
    in_specs=[pl.BlockSpec((1, _B, _E), lambda s: (s, 0, 0))],
    out_specs=pl.BlockSpec((1, _E, _B), lambda s: (s, 0, 0)),
    out_shape=jax.ShapeDtypeStruct((_S, _E, _B), jnp.float32),
)


def kernel(word_idx, table, ngram_idx, ngram_cnt):
    idxp = jnp.pad(ngram_idx, ((0, 0), (0, _GPAD - ngram_idx.shape[1])))
    idxp = idxp.reshape(_NS, _WPS * _GPAD)
    cntb = jnp.broadcast_to(ngram_cnt[:, None], (_V, _E))
    tok_idx = word_idx.T.reshape(_NW, _NCHUNK, _CHUNK)
    _, out_sb = _ngram_embed(table, idxp, cntb, tok_idx)
    out3 = _fmt(out_sb.reshape(_S, _B, _E))
    return out3.transpose(2, 0, 1)


# speedup vs baseline: 2.3791x; 1.1560x over previous
import functools

import jax
import jax.numpy as jnp
from jax import lax
from jax.experimental import pallas as pl
from jax.experimental.pallas import tpu as pltpu
from jax.experimental.pallas import tpu_sc as plsc

_info = plsc.get_sparse_core_info()
_NC, _NS, _L = _info.num_cores, _info.num_subcores, _info.num_lanes
_NW = _NC * _NS

_B = 1024
_S = 20
_V = 64
_E = 64
_GPAD = 8
_TOK = _B * _S
_TPT = _TOK // _NW
_CHUNK = 128
_NCHUNK = _TPT // _CHUNK

_mesh = plsc.VectorSubcoreMesh(core_axis_name="c", subcore_axis_name="s")
_sc_params = pltpu.CompilerParams(use_tc_tiling_on_sc=False)


def _emb_body(blk_ids, lane_ids, *refs):
    tt_blks = refs[:_GPAD]
    cntb_blk, emb_blk, acc = refs[_GPAD], refs[_GPAD + 1], refs[_GPAD + 2]
    w = pl.program_id(0)
    iota = lax.broadcasted_iota(jnp.int32, (_E, 128), 1)
    total = jnp.zeros((_E, 128), jnp.float32)
    for k in range(_GPAD):
        lane = lane_ids[w * _GPAD + k]
        total = total + jnp.where(iota == lane, tt_blks[k][...], 0.0)
    ones = jnp.ones((1, 128), jnp.float32)
    row = lax.dot_general(ones, total, (((1,), (1,)), ((), ())),
                          preferred_element_type=jnp.float32)
    wmask = lax.broadcasted_iota(jnp.int32, (_V, _E), 0) == w
    upd = jnp.where(wmask, jnp.broadcast_to(row, (_V, _E)), 0.0)

    @pl.when(w == 0)
    def _():
        acc[...] = upd

    @pl.when(w > 0)
    def _():
        acc[...] = acc[...] + upd

    @pl.when(w == _V - 1)
    def _():
        emb_blk[...] = acc[...] / cntb_blk[...]


def _make_tt_spec(k):
    return pl.BlockSpec((_E, 128), lambda w, blk, lane, _k=k: (0, blk[w * _GPAD + _k]))


_build_emb = pl.pallas_call(
    _emb_body,
    grid_spec=pltpu.PrefetchScalarGridSpec(
        num_scalar_prefetch=2,
        grid=(_V,),
        in_specs=[_make_tt_spec(k) for k in range(_GPAD)]
        + [pl.BlockSpec((_V, _E), lambda w, blk, lane: (0, 0))],
        out_specs=pl.BlockSpec((_V, _E), lambda w, blk, lane: (0, 0)),
        scratch_shapes=[pltpu.VMEM((_V, _E), jnp.float32)],
    ),
    out_shape=jax.ShapeDtypeStruct((_V, _E), jnp.float32),
)


@functools.partial(
    pl.kernel,
    mesh=_mesh,
    compiler_params=_sc_params,
    out_type=jax.ShapeDtypeStruct((_TOK, _E), jnp.float32),
    scratch_types=[
        pltpu.VMEM((_NCHUNK, _CHUNK), jnp.int32),
        pltpu.VMEM((_TPT, _E), jnp.float32),
        pltpu.SemaphoreType.DMA,
    ],
)
def _expand(emb_hbm, idx_hbm, out_hbm, idx_v, rows_v, sem):
    wid = lax.axis_index("s") * _NC + lax.axis_index("c")
    pltpu.sync_copy(idx_hbm.at[wid], idx_v)
    copies = []
    for j in range(_NCHUNK):
        copies.append(
            pltpu.async_copy(
                emb_hbm.at[idx_v.at[j]],
                rows_v.at[pl.ds(j * _CHUNK, _CHUNK)],
                sem,
            )
        )
    for c in copies:
        c.wait()
    pltpu.sync_copy(rows_v, out_hbm.at[pl.ds(wid * _TPT, _TPT)])


def _fmt_body(src_blk, dst_blk):
    dst_blk[0] = jnp.transpose(src_blk[0], (1, 0))


_fmt = pl.pallas_call(
    _fmt_body,
    grid=(_S,),
    in_specs=[pl.BlockSpec((1, _B, _E), lambda s: (s, 0, 0))],
    out_specs=pl.BlockSpec((1, _E, _B), lambda s: (s, 0, 0)),
    out_shape=jax.ShapeDtypeStruct((_S, _E, _B), jnp.float32),
)


def kernel(word_idx, table, ngram_idx, ngram_cnt):
    tt = table.T
    idxp = jnp.pad(ngram_idx, ((0, 0), (0, _GPAD - ngram_idx.shape[1])))
    idxf = idxp.reshape(_V * _GPAD)
    blk_ids = idxf // 128
    lane_ids = idxf % 128
    cntb = jnp.broadcast_to(ngram_cnt[:, None], (_V, _E))
    emb = _build_emb(blk_ids, lane_ids, *([tt] * _GPAD), cntb)
    tok_idx = word_idx.T.reshape(_NW, _NCHUNK, _CHUNK)
    out_sb = _expand(emb, tok_idx)
    out3 = _fmt(out_sb.reshape(_S, _B, _E))
    return out3.transpose(2, 0, 1)
